# Initial kernel scaffold; baseline (speedup 1.0000x reference)
import jax
import jax.numpy as jnp
from jax.experimental import pallas as pl


def kernel(x, conv1_w_flat, conv1_b, conv2_b, w2_lane, csel, pool_swe, pool_swo, pool_she, pool_sho, fc1_m, dmask, fold, fc1_b, fc2_wt, fc2_b):
    raise NotImplementedError("write your pallas kernel here")


# speedup vs baseline: 5.3402x; 5.3402x over previous
import jax
import jax.numpy as jnp
from jax.experimental import pallas as pl
from jax.experimental.pallas import tpu as pltpu

F32 = jnp.float32
BT = 4


def _shift_lanes_left(a):
    return jnp.concatenate([a[:, 1:], a[:, :1]], axis=1)


def _fused_kernel(x_ref,
                  w1_ref,
                  w2_ref,
                  m1_ref,
                  she_ref, sho_ref,
                  she2_ref, sho2_ref,
                  b1_ref,
                  b2_ref,
                  dmask_ref,
                  fold_ref,
                  fb1_ref, fw2_ref, fb2_ref,
                  o_ref,
                  slab_ref,
                  f_ref):
    for bi in range(BT):
        for ic in range(12):
            slab_ref[:, ic * 98:(ic + 1) * 98] = x_ref[bi, ic]
        acc = jnp.dot(slab_ref[0:72, :], w1_ref[0],
                      preferred_element_type=F32)
        acc = acc + jnp.dot(slab_ref[1:73, :], w1_ref[1],
                            preferred_element_type=F32)
        acc = acc + jnp.dot(slab_ref[2:74, :], w1_ref[2],
                            preferred_element_type=F32)
        act = jnp.maximum(acc + b1_ref[...], 0.0)

        wmax = jnp.maximum(act, _shift_lanes_left(act))
        p1 = jnp.maximum(
            jnp.dot(she_ref[...], wmax, preferred_element_type=F32),
            jnp.dot(sho_ref[...], wmax, preferred_element_type=F32))

        acc2 = jnp.dot(p1[0:34, :], w2_ref[0], preferred_element_type=F32)
        acc2 = acc2 + jnp.dot(p1[1:35, :], w2_ref[1],
                              preferred_element_type=F32)
        acc2 = acc2 + jnp.dot(p1[2:36, :], w2_ref[2],
                              preferred_element_type=F32)
        act2 = jnp.maximum(acc2 + b2_ref[...], 0.0)

        wmax2 = jnp.maximum(act2, _shift_lanes_left(act2))
        p2 = jnp.maximum(
            jnp.dot(she2_ref[...], wmax2, preferred_element_type=F32),
            jnp.dot(sho2_ref[...], wmax2, preferred_element_type=F32))

        g = jnp.dot(p2, m1_ref[...], preferred_element_type=F32)
        f_ref[bi:bi + 1, :] = jnp.sum(g * dmask_ref[...], axis=0,
                                      keepdims=True)

    h = jnp.maximum(jnp.dot(f_ref[...], fold_ref[...],
                            preferred_element_type=F32) + fb1_ref[...], 0.0)
    o_ref[0] = jnp.dot(h, fw2_ref[...],
                       preferred_element_type=F32) + fb2_ref[...]


def kernel(x, conv1_w_flat, conv1_b, conv2_b, w2_lane, csel,
           pool_swe, pool_swo, pool_she, pool_sho,
           fc1_m, dmask, fold, fc1_b, fc2_wt, fc2_b):
    B = x.shape[0]

    cw1 = conv1_w_flat.reshape(8, 12, 3, 3)
    wp = jnp.arange(98)[:, None]
    w = jnp.arange(96)[None, :]
    s1 = jnp.stack([(wp == w + kw).astype(F32) for kw in range(3)])
    w1t = jnp.einsum('kpw,oihk->hipow', s1, cw1)
    w1t = w1t.reshape(3, 12 * 98, 8 * 96)

    cw2 = w2_lane.reshape(4, 9, 8, 48)[:, :, :, 0].reshape(4, 3, 3, 8)
    cw2 = cw2.transpose(0, 3, 1, 2)
    u = jnp.arange(96)[:, None]
    w2c = jnp.arange(46)[None, :]
    s2 = jnp.stack([(u == 2 * (w2c + kw)).astype(F32) for kw in range(3)])
    w2t = jnp.einsum('kuw,oihk->hiuow', s2, cw2)
    w2t = w2t.reshape(3, 8 * 96, 4 * 46)

    m1p = jnp.zeros((4, 46, 544), F32).at[:, 0::2, :].set(fc1_m)
    m1p = m1p.reshape(184, 544)

    b1row = jnp.repeat(conv1_b, 96).reshape(1, 768)
    b2row = jnp.repeat(conv2_b, 46).reshape(1, 184)

    she2 = pool_she[:17, :34]
    sho2 = pool_sho[:17, :34]

    grid = (B // BT,)
    out = pl.pallas_call(
        _fused_kernel,
        out_shape=jax.ShapeDtypeStruct((B // BT, BT, 24), F32),
        grid=grid,
        in_specs=[
            pl.BlockSpec((BT, 12, 74, 98), lambda b: (b, 0, 0, 0)),
            pl.BlockSpec((3, 1176, 768), lambda b: (0, 0, 0)),
            pl.BlockSpec((3, 768, 184), lambda b: (0, 0, 0)),
            pl.BlockSpec((184, 544), lambda b: (0, 0)),
            pl.BlockSpec((36, 72), lambda b: (0, 0)),
            pl.BlockSpec((36, 72), lambda b: (0, 0)),
            pl.BlockSpec((17, 34), lambda b: (0, 0)),
            pl.BlockSpec((17, 34), lambda b: (0, 0)),
            pl.BlockSpec((1, 768), lambda b: (0, 0)),
            pl.BlockSpec((1, 184), lambda b: (0, 0)),
            pl.BlockSpec((17, 544), lambda b: (0, 0)),
            pl.BlockSpec((544, 32), lambda b: (0, 0)),
            pl.BlockSpec((1, 32), lambda b: (0, 0)),
            pl.BlockSpec((32, 24), lambda b: (0, 0)),
            pl.BlockSpec((1, 24), lambda b: (0, 0)),
        ],
        out_specs=pl.BlockSpec((1, BT, 24), lambda b: (b, 0, 0)),
        scratch_shapes=[
            pltpu.VMEM((74, 1176), F32),
            pltpu.VMEM((BT, 544), F32),
        ],
        compiler_params=pltpu.CompilerParams(
            dimension_semantics=("parallel",)),
    )(x, w1t, w2t, m1p, pool_she, pool_sho, she2, sho2,
      b1row, b2row, dmask, fold, fc1_b, fc2_wt, fc2_b)
    return out.reshape(B, 24)


# speedup vs baseline: 7.6558x; 1.4336x over previous
import jax
import jax.numpy as jnp
from jax.experimental import pallas as pl
from jax.experimental.pallas import tpu as pltpu

F32 = jnp.float32
BF16 = jnp.bfloat16
BT = 8
NR = BT * 74
M1 = NR - 2
NP1 = BT * 36
M2 = NP1 - 2
NP2 = BT * 17


def _shift_lanes_left(a):
    return jnp.concatenate([a[:, 1:], a[:, :1]], axis=1)


def _fused_kernel(x_ref,
                  w1_ref,
                  w2_ref,
                  m1_ref,
                  pe1_ref, po1_ref,
                  pe2_ref, po2_ref,
                  seg_ref,
                  b1_ref,
                  b2_ref,
                  dmask_ref,
                  fold_ref,
                  fb1_ref, fw2_ref, fb2_ref,
                  o_ref,
                  slab_ref,
                  wm1_ref,
                  p1_ref,
                  wm2_ref):
    for bi in range(BT):
        for ic in range(12):
            slab_ref[bi * 74:(bi + 1) * 74, ic * 98:(ic + 1) * 98] = (
                x_ref[bi, ic].astype(BF16))

    acc = jnp.dot(slab_ref[0:M1, :], w1_ref[0], preferred_element_type=F32)
    acc = acc + jnp.dot(slab_ref[1:M1 + 1, :], w1_ref[1],
                        preferred_element_type=F32)
    acc = acc + jnp.dot(slab_ref[2:M1 + 2, :], w1_ref[2],
                        preferred_element_type=F32)
    act = jnp.maximum(acc + b1_ref[...], 0.0)
    wm1_ref[0:M1, :] = jnp.maximum(act, _shift_lanes_left(act)).astype(BF16)

    p1 = jnp.maximum(
        jnp.dot(pe1_ref[...], wm1_ref[0:M1, :], preferred_element_type=F32),
        jnp.dot(po1_ref[...], wm1_ref[0:M1, :], preferred_element_type=F32))
    p1_ref[...] = p1.astype(BF16)

    acc2 = jnp.dot(p1_ref[0:M2, :], w2_ref[0], preferred_element_type=F32)
    acc2 = acc2 + jnp.dot(p1_ref[1:M2 + 1, :], w2_ref[1],
                          preferred_element_type=F32)
    acc2 = acc2 + jnp.dot(p1_ref[2:M2 + 2, :], w2_ref[2],
                          preferred_element_type=F32)
    act2 = jnp.maximum(acc2 + b2_ref[...], 0.0)
    wm2_ref[0:M2, :] = jnp.maximum(act2, _shift_lanes_left(act2)).astype(BF16)

    p2 = jnp.maximum(
        jnp.dot(pe2_ref[...], wm2_ref[0:M2, :], preferred_element_type=F32),
        jnp.dot(po2_ref[...], wm2_ref[0:M2, :], preferred_element_type=F32))
    g = jnp.dot(p2.astype(BF16), m1_ref[...], preferred_element_type=F32)
    masked = (g * dmask_ref[...]).astype(BF16)
    f = jnp.dot(seg_ref[...], masked, preferred_element_type=F32)

    h = jnp.maximum(jnp.dot(f.astype(BF16), fold_ref[...],
                            preferred_element_type=F32) + fb1_ref[...], 0.0)
    o_ref[0] = jnp.dot(h.astype(BF16), fw2_ref[...],
                       preferred_element_type=F32) + fb2_ref[...]


def kernel(x, conv1_w_flat, conv1_b, conv2_b, w2_lane, csel,
           pool_swe, pool_swo, pool_she, pool_sho,
           fc1_m, dmask, fold, fc1_b, fc2_wt, fc2_b):
    B = x.shape[0]

    cw1 = conv1_w_flat.reshape(8, 12, 3, 3)
    wp = jnp.arange(98)[:, None]
    w = jnp.arange(96)[None, :]
    s1 = jnp.stack([(wp == w + kw).astype(F32) for kw in range(3)])
    w1t = jnp.einsum('kpw,oihk->hipow', s1, cw1).reshape(3, 1176, 768)

    cw2 = w2_lane.reshape(4, 9, 8, 48)[:, :, :, 0].reshape(4, 3, 3, 8)
    cw2 = cw2.transpose(0, 3, 1, 2)
    u = jnp.arange(96)[:, None]
    w2c = jnp.arange(46)[None, :]
    s2 = jnp.stack([(u == 2 * (w2c + kw)).astype(F32) for kw in range(3)])
    w2t = jnp.einsum('kuw,oihk->hiuow', s2, cw2).reshape(3, 768, 184)

    m1p = jnp.zeros((4, 46, 544), F32).at[:, 0::2, :].set(fc1_m)
    m1p = m1p.reshape(184, 544)

    i1 = jnp.arange(NP1)
    c1 = 74 * (i1 // 36) + 2 * (i1 % 36)
    cols1 = jnp.arange(M1)[None, :]
    pe1 = (cols1 == c1[:, None]).astype(BF16)
    po1 = (cols1 == (c1 + 1)[:, None]).astype(BF16)
    i2 = jnp.arange(NP2)
    c2 = 36 * (i2 // 17) + 2 * (i2 % 17)
    cols2 = jnp.arange(M2)[None, :]
    pe2 = (cols2 == c2[:, None]).astype(BF16)
    po2 = (cols2 == (c2 + 1)[:, None]).astype(BF16)
    seg = (jnp.arange(NP2)[None, :] // 17
           == jnp.arange(BT)[:, None]).astype(BF16)

    b1row = jnp.repeat(conv1_b, 96).reshape(1, 768)
    b2row = jnp.repeat(conv2_b, 46).reshape(1, 184)
    dmask_big = jnp.tile(dmask, (BT, 1))

    grid = (B // BT,)
    out = pl.pallas_call(
        _fused_kernel,
        out_shape=jax.ShapeDtypeStruct((B // BT, BT, 24), F32),
        grid=grid,
        in_specs=[
            pl.BlockSpec((BT, 12, 74, 98), lambda b: (b, 0, 0, 0)),
            pl.BlockSpec((3, 1176, 768), lambda b: (0, 0, 0)),
            pl.BlockSpec((3, 768, 184), lambda b: (0, 0, 0)),
            pl.BlockSpec((184, 544), lambda b: (0, 0)),
            pl.BlockSpec((NP1, M1), lambda b: (0, 0)),
            pl.BlockSpec((NP1, M1), lambda b: (0, 0)),
            pl.BlockSpec((NP2, M2), lambda b: (0, 0)),
            pl.BlockSpec((NP2, M2), lambda b: (0, 0)),
            pl.BlockSpec((BT, NP2), lambda b: (0, 0)),
            pl.BlockSpec((1, 768), lambda b: (0, 0)),
            pl.BlockSpec((1, 184), lambda b: (0, 0)),
            pl.BlockSpec((NP2, 544), lambda b: (0, 0)),
            pl.BlockSpec((544, 32), lambda b: (0, 0)),
            pl.BlockSpec((1, 32), lambda b: (0, 0)),
            pl.BlockSpec((32, 24), lambda b: (0, 0)),
            pl.BlockSpec((1, 24), lambda b: (0, 0)),
        ],
        out_specs=pl.BlockSpec((1, BT, 24), lambda b: (b, 0, 0)),
        scratch_shapes=[
            pltpu.VMEM((NR, 1176), BF16),
            pltpu.VMEM((NR, 768), BF16),
            pltpu.VMEM((NP1, 768), BF16),
            pltpu.VMEM((NP1, 184), BF16),
        ],
        compiler_params=pltpu.CompilerParams(
            dimension_semantics=("parallel",)),
    )(x, w1t.astype(BF16), w2t.astype(BF16), m1p.astype(BF16),
      pe1, po1, pe2, po2, seg, b1row, b2row, dmask_big,
      fold.astype(BF16), fc1_b, fc2_wt.astype(BF16), fc2_b)
    return out.reshape(B, 24)
